# baseline (speedup 1.0000x reference)
import jax
import jax.numpy as jnp
from jax.experimental import pallas as pl
from jax.experimental.pallas import tpu as pltpu

_M = 100000
_L = 64
_S = 256
_NCHUNK = 10
_ROWS = _M // _NCHUNK
_SMALL = (_M // 8, 8)


def _tree_add_body(scal_ref, prob_ref,
                   seq_in, bel_in, row_seq, row_bel, sl_in, pr_in,
                   seq_out, bel_out, sl_out, pr_out,
                   seq_sems, bel_sems, win_seq, win_bel, wsem):
    sz = scal_ref[0]
    base = pl.multiple_of((sz // 8) * 8, 8)
    r = sz - base
    pltpu.make_async_copy(seq_in.at[pl.ds(base, 8)], win_seq, wsem).start()
    pltpu.make_async_copy(bel_in.at[pl.ds(base, 8)], win_bel, wsem).start()
    for c in range(_NCHUNK):
        pltpu.make_async_copy(seq_in.at[pl.ds(c * _ROWS, _ROWS)],
                              seq_out.at[pl.ds(c * _ROWS, _ROWS)],
                              seq_sems.at[c]).start()
        pltpu.make_async_copy(bel_in.at[pl.ds(c * _ROWS, _ROWS)],
                              bel_out.at[pl.ds(c * _ROWS, _ROWS)],
                              bel_sems.at[c]).start()
    flat = (jax.lax.broadcasted_iota(jnp.int32, _SMALL, 0) * 8
            + jax.lax.broadcasted_iota(jnp.int32, _SMALL, 1))
    sl_out[...] = jnp.where(flat == sz, scal_ref[1], sl_in[...])
    pr_out[...] = jnp.where(flat == sz, prob_ref[0], pr_in[...])
    pltpu.make_async_copy(seq_in.at[pl.ds(base, 8)], win_seq, wsem).wait()
    pltpu.make_async_copy(bel_in.at[pl.ds(base, 8)], win_bel, wsem).wait()
    ri_seq = jax.lax.broadcasted_iota(jnp.int32, (8, _L), 0)
    ri_bel = jax.lax.broadcasted_iota(jnp.int32, (8, _S), 0)
    win_seq[...] = jnp.where(ri_seq == r, row_seq[...], win_seq[...])
    win_bel[...] = jnp.where(ri_bel == r, row_bel[...], win_bel[...])
    for c in range(_NCHUNK):
        pltpu.make_async_copy(seq_in.at[pl.ds(c * _ROWS, _ROWS)],
                              seq_out.at[pl.ds(c * _ROWS, _ROWS)],
                              seq_sems.at[c]).wait()
        pltpu.make_async_copy(bel_in.at[pl.ds(c * _ROWS, _ROWS)],
                              bel_out.at[pl.ds(c * _ROWS, _ROWS)],
                              bel_sems.at[c]).wait()
    pltpu.make_async_copy(win_seq, seq_out.at[pl.ds(base, 8)], wsem).start()
    pltpu.make_async_copy(win_bel, bel_out.at[pl.ds(base, 8)], wsem).start()
    pltpu.make_async_copy(win_seq, seq_out.at[pl.ds(base, 8)], wsem).wait()
    pltpu.make_async_copy(win_bel, bel_out.at[pl.ds(base, 8)], wsem).wait()


def kernel(sequences, sequence_lengths, belief_states, probabilities, size,
           sequence, sequence_length, belief_state, probability):
    sz = jnp.asarray(size, jnp.int32)
    scal = jnp.stack([sz, jnp.asarray(sequence_length, jnp.int32)])
    prob = jnp.reshape(jnp.asarray(probability, jnp.float32), (1,))
    row_seq = jnp.reshape(jnp.asarray(sequence, jnp.int32), (1, _L))
    row_bel = jnp.reshape(jnp.asarray(belief_state, jnp.float32), (1, _S))
    sl2 = jnp.reshape(sequence_lengths, _SMALL)
    pr2 = jnp.reshape(probabilities, _SMALL)

    hbm = pl.BlockSpec(memory_space=pltpu.MemorySpace.HBM)
    smem = pl.BlockSpec(memory_space=pltpu.MemorySpace.SMEM)
    vmem = pl.BlockSpec(memory_space=pltpu.MemorySpace.VMEM)

    seq_o, bel_o, sl_o, pr_o = pl.pallas_call(
        _tree_add_body,
        out_shape=(
            jax.ShapeDtypeStruct((_M, _L), jnp.int32),
            jax.ShapeDtypeStruct((_M, _S), jnp.float32),
            jax.ShapeDtypeStruct(_SMALL, jnp.int32),
            jax.ShapeDtypeStruct(_SMALL, jnp.float32),
        ),
        in_specs=[smem, smem, hbm, hbm, vmem, vmem, vmem, vmem],
        out_specs=(hbm, hbm, vmem, vmem),
        scratch_shapes=[
            pltpu.SemaphoreType.DMA((_NCHUNK,)),
            pltpu.SemaphoreType.DMA((_NCHUNK,)),
            pltpu.VMEM((8, _L), jnp.int32),
            pltpu.VMEM((8, _S), jnp.float32),
            pltpu.SemaphoreType.DMA,
        ],
    )(scal, prob, sequences, belief_states, row_seq, row_bel, sl2, pr2)

    return (seq_o, jnp.reshape(sl_o, (_M,)), bel_o, jnp.reshape(pr_o, (_M,)),
            sz + 1)


# speedup vs baseline: 23.0196x; 23.0196x over previous
import jax
import jax.numpy as jnp
from jax.experimental import pallas as pl
from jax.experimental.pallas import tpu as pltpu

_M = 100000
_L = 64
_S = 256
_NB = 50
_RB = _M // _NB
_SMALL = (_M // 8, 8)


def _tree_add_body(scal_ref, prob_ref,
                   seq_in, bel_in, row_seq, row_bel, sl_in, pr_in,
                   seq_out, bel_out, sl_out, pr_out):
    i = pl.program_id(0)
    sz = scal_ref[0]
    blk = sz // _RB
    r = sz - blk * _RB

    seq_out[...] = seq_in[...]
    bel_out[...] = bel_in[...]

    @pl.when(i == blk)
    def _patch_rows():
        seq_out[pl.ds(r, 1), :] = row_seq[...]
        bel_out[pl.ds(r, 1), :] = row_bel[...]

    @pl.when(i == 0)
    def _patch_small():
        flat = (jax.lax.broadcasted_iota(jnp.int32, _SMALL, 0) * 8
                + jax.lax.broadcasted_iota(jnp.int32, _SMALL, 1))
        sl_out[...] = jnp.where(flat == sz, scal_ref[1], sl_in[...])
        pr_out[...] = jnp.where(flat == sz, prob_ref[0], pr_in[...])


def kernel(sequences, sequence_lengths, belief_states, probabilities, size,
           sequence, sequence_length, belief_state, probability):
    sz = jnp.asarray(size, jnp.int32)
    scal = jnp.stack([sz, jnp.asarray(sequence_length, jnp.int32)])
    prob = jnp.reshape(jnp.asarray(probability, jnp.float32), (1,))
    row_seq = jnp.reshape(jnp.asarray(sequence, jnp.int32), (1, _L))
    row_bel = jnp.reshape(jnp.asarray(belief_state, jnp.float32), (1, _S))
    sl2 = jnp.reshape(sequence_lengths, _SMALL)
    pr2 = jnp.reshape(probabilities, _SMALL)

    smem = pl.BlockSpec(memory_space=pltpu.MemorySpace.SMEM)
    const2 = lambda i: (0, 0)

    seq_o, bel_o, sl_o, pr_o = pl.pallas_call(
        _tree_add_body,
        grid=(_NB,),
        out_shape=(
            jax.ShapeDtypeStruct((_M, _L), jnp.int32),
            jax.ShapeDtypeStruct((_M, _S), jnp.float32),
            jax.ShapeDtypeStruct(_SMALL, jnp.int32),
            jax.ShapeDtypeStruct(_SMALL, jnp.float32),
        ),
        in_specs=[smem, smem,
                  pl.BlockSpec((_RB, _L), lambda i: (i, 0)),
                  pl.BlockSpec((_RB, _S), lambda i: (i, 0)),
                  pl.BlockSpec((1, _L), const2),
                  pl.BlockSpec((1, _S), const2),
                  pl.BlockSpec(_SMALL, const2),
                  pl.BlockSpec(_SMALL, const2)],
        out_specs=(pl.BlockSpec((_RB, _L), lambda i: (i, 0)),
                   pl.BlockSpec((_RB, _S), lambda i: (i, 0)),
                   pl.BlockSpec(_SMALL, const2),
                   pl.BlockSpec(_SMALL, const2)),
    )(scal, prob, sequences, belief_states, row_seq, row_bel, sl2, pr2)

    return (seq_o, jnp.reshape(sl_o, (_M,)), bel_o, jnp.reshape(pr_o, (_M,)),
            sz + 1)


# speedup vs baseline: 23.5372x; 1.0225x over previous
import jax
import jax.numpy as jnp
from jax.experimental import pallas as pl
from jax.experimental.pallas import tpu as pltpu

_M = 100000
_L = 64
_S = 256
_NS = 10
_RS = _M // _NS
_KS = 4
_NB = 20
_RB = _M // _NB
_KB = 4
_SMALL = (8, _M // 8)


def _tree_add_body(scal_ref, prob_ref,
                   seq_in, bel_in, row_seq, row_bel, sl_in, pr_in,
                   seq_out, bel_out, sl_out, pr_out,
                   tile_seq, tile_bel, *scratch):
    slots_seq = list(scratch[:_KS])
    slots_bel = list(scratch[_KS:_KS + _KB])
    sem_si, sem_so, sem_bi, sem_bo, sem_p1, sem_p2 = scratch[_KS + _KB:]

    sz = scal_ref[0]
    t = pl.multiple_of((sz // 8) * 8, 8)
    r = sz - t

    seq_ins = [pltpu.make_async_copy(seq_in.at[pl.ds(c * _RS, _RS), :],
                                     slots_seq[c % _KS], sem_si.at[c % _KS])
               for c in range(_NS)]
    seq_outs = [pltpu.make_async_copy(slots_seq[c % _KS],
                                      seq_out.at[pl.ds(c * _RS, _RS), :],
                                      sem_so.at[c % _KS])
                for c in range(_NS)]
    bel_ins = [pltpu.make_async_copy(bel_in.at[pl.ds(c * _RB, _RB), :],
                                     slots_bel[c % _KB], sem_bi.at[c % _KB])
               for c in range(_NB)]
    bel_outs = [pltpu.make_async_copy(slots_bel[c % _KB],
                                      bel_out.at[pl.ds(c * _RB, _RB), :],
                                      sem_bo.at[c % _KB])
                for c in range(_NB)]

    t1 = pltpu.make_async_copy(seq_in.at[pl.ds(t, 8), :], tile_seq, sem_p1)
    t1.start()
    t2 = pltpu.make_async_copy(bel_in.at[pl.ds(t, 8), :], tile_bel, sem_p2)
    t2.start()

    for c in range(_KS):
        seq_ins[c].start()
    for c in range(_KB):
        bel_ins[c].start()

    flat = (jax.lax.broadcasted_iota(jnp.int32, _SMALL, 0) * (_M // 8)
            + jax.lax.broadcasted_iota(jnp.int32, _SMALL, 1))
    sl_out[...] = jnp.where(flat == sz, scal_ref[1], sl_in[...])
    pr_out[...] = jnp.where(flat == sz, prob_ref[0], pr_in[...])

    t1.wait()
    row0 = jax.lax.broadcasted_iota(jnp.int32, (8, _L), 0)
    tile_seq[...] = jnp.where(row0 == r, row_seq[...], tile_seq[...])
    t2.wait()
    row1 = jax.lax.broadcasted_iota(jnp.int32, (8, _S), 0)
    tile_bel[...] = jnp.where(row1 == r, row_bel[...], tile_bel[...])

    def seq_step(c):
        if c >= _KS:
            seq_outs[c - _KS].wait()
            seq_ins[c].start()
        seq_ins[c].wait()
        seq_outs[c].start()

    def bel_step(c):
        if c >= _KB:
            bel_outs[c - _KB].wait()
            bel_ins[c].start()
        bel_ins[c].wait()
        bel_outs[c].start()

    for i in range(_NS):
        seq_step(i)
        bel_step(2 * i)
        bel_step(2 * i + 1)

    for c in range(_NS - _KS, _NS):
        seq_outs[c].wait()
    for c in range(_NB - _KB, _NB):
        bel_outs[c].wait()

    p1 = pltpu.make_async_copy(tile_seq, seq_out.at[pl.ds(t, 8), :], sem_p1)
    p1.start()
    p2 = pltpu.make_async_copy(tile_bel, bel_out.at[pl.ds(t, 8), :], sem_p2)
    p2.start()
    p1.wait()
    p2.wait()


def kernel(sequences, sequence_lengths, belief_states, probabilities, size,
           sequence, sequence_length, belief_state, probability):
    sz = jnp.asarray(size, jnp.int32)
    scal = jnp.stack([sz, jnp.asarray(sequence_length, jnp.int32)])
    prob = jnp.reshape(jnp.asarray(probability, jnp.float32), (1,))
    row_seq = jnp.reshape(jnp.asarray(sequence, jnp.int32), (1, _L))
    row_bel = jnp.reshape(jnp.asarray(belief_state, jnp.float32), (1, _S))
    sl2 = jnp.reshape(sequence_lengths, _SMALL)
    pr2 = jnp.reshape(probabilities, _SMALL)

    smem = pl.BlockSpec(memory_space=pltpu.MemorySpace.SMEM)
    anym = pl.BlockSpec(memory_space=pltpu.MemorySpace.HBM)
    vmem = pl.BlockSpec(memory_space=pltpu.MemorySpace.VMEM)

    scratch = ([pltpu.VMEM((8, _L), jnp.int32),
                pltpu.VMEM((8, _S), jnp.float32)]
               + [pltpu.VMEM((_RS, _L), jnp.int32) for _ in range(_KS)]
               + [pltpu.VMEM((_RB, _S), jnp.float32) for _ in range(_KB)]
               + [pltpu.SemaphoreType.DMA((_KS,)),
                  pltpu.SemaphoreType.DMA((_KS,)),
                  pltpu.SemaphoreType.DMA((_KB,)),
                  pltpu.SemaphoreType.DMA((_KB,)),
                  pltpu.SemaphoreType.DMA,
                  pltpu.SemaphoreType.DMA])

    seq_o, bel_o, sl_o, pr_o = pl.pallas_call(
        _tree_add_body,
        out_shape=(
            jax.ShapeDtypeStruct((_M, _L), jnp.int32),
            jax.ShapeDtypeStruct((_M, _S), jnp.float32),
            jax.ShapeDtypeStruct(_SMALL, jnp.int32),
            jax.ShapeDtypeStruct(_SMALL, jnp.float32),
        ),
        in_specs=[smem, smem, anym, anym, vmem, vmem, vmem, vmem],
        out_specs=(anym, anym, vmem, vmem),
        scratch_shapes=scratch,
    )(scal, prob, sequences, belief_states, row_seq, row_bel, sl2, pr2)

    return (seq_o, jnp.reshape(sl_o, (_M,)), bel_o, jnp.reshape(pr_o, (_M,)),
            sz + 1)


# speedup vs baseline: 24.5243x; 1.0419x over previous
import functools

import jax
import jax.numpy as jnp
from jax import lax
from jax.experimental import pallas as pl
from jax.experimental.pallas import tpu as pltpu
from jax.experimental.pallas import tpu_sc as plsc

_M = 100000
_L = 64
_S = 256
_SMALL = (8, _M // 8)

_NS = 10
_RS = _M // _NS
_KS = 4

_NW = 32
_WROWS = 3200
_CROWS = 200
_NCH = _WROWS // _CROWS


def _sc_bel_copy_body(bel_in, bel_out, b0, b1, si0, si1, so0, so1):
    cid = lax.axis_index("c")
    sid = lax.axis_index("s")
    w = sid * 2 + cid
    base = w * _WROWS
    bufs = (b0, b1)
    sin = (si0, si1)
    sout = (so0, so1)

    starts = [jnp.minimum(base + j * _CROWS, _M - _CROWS) for j in range(_NCH)]
    ins = [pltpu.make_async_copy(bel_in.at[pl.ds(starts[j], _CROWS), :],
                                 bufs[j % 2], sin[j % 2])
           for j in range(_NCH)]
    outs = [pltpu.make_async_copy(bufs[j % 2],
                                  bel_out.at[pl.ds(starts[j], _CROWS), :],
                                  sout[j % 2])
            for j in range(_NCH)]

    for j in range(_NCH):
        if j >= 2:
            outs[j - 2].wait()
        ins[j].start()
        ins[j].wait()
        outs[j].start()
    outs[_NCH - 2].wait()
    outs[_NCH - 1].wait()


_sc_bel_copy = functools.partial(
    pl.kernel,
    out_type=jax.ShapeDtypeStruct((_M, _S), jnp.float32),
    mesh=plsc.VectorSubcoreMesh(core_axis_name="c", subcore_axis_name="s"),
    scratch_types=[
        pltpu.VMEM((_CROWS, _S), jnp.float32),
        pltpu.VMEM((_CROWS, _S), jnp.float32),
        pltpu.SemaphoreType.DMA,
        pltpu.SemaphoreType.DMA,
        pltpu.SemaphoreType.DMA,
        pltpu.SemaphoreType.DMA,
    ],
)(_sc_bel_copy_body)


def _tc_main_body(scal_ref, prob_ref, row_seq, seq_in, sl_in, pr_in,
                  seq_out, sl_out, pr_out,
                  tile_seq, *scratch):
    slots_seq = list(scratch[:_KS])
    sem_si, sem_so, sem_p1 = scratch[_KS:]

    sz = scal_ref[0]
    t = pl.multiple_of((sz // 8) * 8, 8)
    r = sz - t

    seq_ins = [pltpu.make_async_copy(seq_in.at[pl.ds(c * _RS, _RS), :],
                                     slots_seq[c % _KS], sem_si.at[c % _KS])
               for c in range(_NS)]
    seq_outs = [pltpu.make_async_copy(slots_seq[c % _KS],
                                      seq_out.at[pl.ds(c * _RS, _RS), :],
                                      sem_so.at[c % _KS])
                for c in range(_NS)]

    t1 = pltpu.make_async_copy(seq_in.at[pl.ds(t, 8), :], tile_seq, sem_p1)
    t1.start()

    for c in range(_KS):
        seq_ins[c].start()

    flat = (jax.lax.broadcasted_iota(jnp.int32, _SMALL, 0) * (_M // 8)
            + jax.lax.broadcasted_iota(jnp.int32, _SMALL, 1))
    sl_out[...] = jnp.where(flat == sz, scal_ref[1], sl_in[...])
    pr_out[...] = jnp.where(flat == sz, prob_ref[0], pr_in[...])

    t1.wait()
    row0 = jax.lax.broadcasted_iota(jnp.int32, (8, _L), 0)
    tile_seq[...] = jnp.where(row0 == r, row_seq[...], tile_seq[...])

    for c in range(_NS):
        if c >= _KS:
            seq_outs[c - _KS].wait()
            seq_ins[c].start()
        seq_ins[c].wait()
        seq_outs[c].start()
    for c in range(_NS - _KS, _NS):
        seq_outs[c].wait()

    p1 = pltpu.make_async_copy(tile_seq, seq_out.at[pl.ds(t, 8), :], sem_p1)
    p1.start()
    p1.wait()


def _tc_bel_patch_body(scal_ref, row_bel, bel_io, bel_out, tile_bel, sem):
    sz = scal_ref[0]
    t = pl.multiple_of((sz // 8) * 8, 8)
    r = sz - t
    t2 = pltpu.make_async_copy(bel_io.at[pl.ds(t, 8), :], tile_bel, sem)
    t2.start()
    t2.wait()
    row1 = jax.lax.broadcasted_iota(jnp.int32, (8, _S), 0)
    tile_bel[...] = jnp.where(row1 == r, row_bel[...], tile_bel[...])
    p2 = pltpu.make_async_copy(tile_bel, bel_out.at[pl.ds(t, 8), :], sem)
    p2.start()
    p2.wait()


def kernel(sequences, sequence_lengths, belief_states, probabilities, size,
           sequence, sequence_length, belief_state, probability):
    sz = jnp.asarray(size, jnp.int32)
    scal = jnp.stack([sz, jnp.asarray(sequence_length, jnp.int32)])
    prob = jnp.reshape(jnp.asarray(probability, jnp.float32), (1,))
    row_seq = jnp.reshape(jnp.asarray(sequence, jnp.int32), (1, _L))
    row_bel = jnp.reshape(jnp.asarray(belief_state, jnp.float32), (1, _S))
    sl2 = jnp.reshape(sequence_lengths, _SMALL)
    pr2 = jnp.reshape(probabilities, _SMALL)

    smem = pl.BlockSpec(memory_space=pltpu.MemorySpace.SMEM)
    anym = pl.BlockSpec(memory_space=pltpu.MemorySpace.HBM)
    vmem = pl.BlockSpec(memory_space=pltpu.MemorySpace.VMEM)

    bel_copied = _sc_bel_copy(belief_states)

    seq_o, sl_o, pr_o = pl.pallas_call(
        _tc_main_body,
        out_shape=(
            jax.ShapeDtypeStruct((_M, _L), jnp.int32),
            jax.ShapeDtypeStruct(_SMALL, jnp.int32),
            jax.ShapeDtypeStruct(_SMALL, jnp.float32),
        ),
        in_specs=[smem, smem, vmem, anym, vmem, vmem],
        out_specs=(anym, vmem, vmem),
        scratch_shapes=([pltpu.VMEM((8, _L), jnp.int32)]
                        + [pltpu.VMEM((_RS, _L), jnp.int32)
                           for _ in range(_KS)]
                        + [pltpu.SemaphoreType.DMA((_KS,)),
                           pltpu.SemaphoreType.DMA((_KS,)),
                           pltpu.SemaphoreType.DMA]),
    )(scal, prob, row_seq, sequences, sl2, pr2)

    bel_o = pl.pallas_call(
        _tc_bel_patch_body,
        out_shape=jax.ShapeDtypeStruct((_M, _S), jnp.float32),
        in_specs=[smem, vmem, anym],
        out_specs=anym,
        input_output_aliases={2: 0},
        scratch_shapes=[pltpu.VMEM((8, _S), jnp.float32),
                        pltpu.SemaphoreType.DMA],
    )(scal, row_bel, bel_copied)

    return (seq_o, jnp.reshape(sl_o, (_M,)), bel_o, jnp.reshape(pr_o, (_M,)),
            sz + 1)
